# baseline (speedup 1.0000x reference)
import functools

import jax
import jax.numpy as jnp
from jax import lax
from jax.experimental import pallas as pl
from jax.experimental.pallas import tpu as pltpu
from jax.experimental.pallas import tpu_sc as plsc


def _sc_gather_time_major(table, idx_tm):
    n_rows, d = idx_tm.shape[0], table.shape[1]
    info = plsc.get_sparse_core_info()
    nc, ns = info.num_cores, info.num_subcores
    nw = nc * ns
    per_w = n_rows // nw
    chunk = 200
    n_iter = per_w // chunk
    mesh = plsc.VectorSubcoreMesh(core_axis_name="c", subcore_axis_name="s")

    @functools.partial(
        pl.kernel,
        mesh=mesh,
        out_type=jax.ShapeDtypeStruct((n_rows, d), jnp.float32),
        scratch_types=[
            pltpu.VMEM((per_w,), jnp.int32),
            pltpu.VMEM((chunk, d), jnp.float32),
            pltpu.VMEM((chunk, d), jnp.float32),
            pltpu.SemaphoreType.DMA,
            pltpu.SemaphoreType.DMA,
            pltpu.SemaphoreType.DMA,
            pltpu.SemaphoreType.DMA,
        ],
    )
    def gather_k(table_hbm, idx_hbm, out_hbm, idx_v, rows0, rows1,
                 g0, g1, w0, w1):
        wid = lax.axis_index("s") * nc + lax.axis_index("c")
        base = wid * per_w
        rows = (rows0, rows1)
        gsem = (g0, g1)
        wsem = (w0, w1)

        pltpu.sync_copy(idx_hbm.at[pl.ds(base, per_w)], idx_v)
        hg = [None, None]
        hw = [None, None]
        hg[0] = pltpu.async_copy(table_hbm.at[idx_v.at[pl.ds(0, chunk)]],
                                 rows[0], gsem[0])
        for i in range(n_iter):
            s = i % 2
            o = 1 - s
            if i + 1 < n_iter:
                if hw[o] is not None:
                    hw[o].wait()
                hg[o] = pltpu.async_copy(
                    table_hbm.at[idx_v.at[pl.ds((i + 1) * chunk, chunk)]],
                    rows[o], gsem[o])
            hg[s].wait()
            hw[s] = pltpu.async_copy(
                rows[s], out_hbm.at[pl.ds(base + i * chunk, chunk)], wsem[s])
        if n_iter > 1:
            hw[(n_iter - 2) % 2].wait()
        hw[(n_iter - 1) % 2].wait()

    return gather_k(table, idx_tm)


def _tc_bilstm_classifier(x3, W_f, U_f, b_f, W_b, U_b, b_b, W1, b1, W2, b2,
                          block_b=512):
    t_len, b_full, d = x3.shape
    u = U_f.shape[0]
    h4 = U_f.shape[1]
    f_dim = W1.shape[1]
    c_dim = W2.shape[1]
    n_blocks = b_full // block_b

    def dot(a, b):
        return jnp.dot(a, b, preferred_element_type=jnp.float32)

    def body(x_ref, wf_ref, bf_ref, wb_ref, bb_ref,
             w1_ref, b1_ref, w2_ref, b2_ref, out_ref, hcat_ref):
        wf = wf_ref[...]
        bf = bf_ref[...]
        wb = wb_ref[...]
        bb = bb_ref[...]

        def sigm(z):
            return 0.5 + 0.5 * jnp.tanh(0.5 * z)

        def step(xt, h, c, wu, bias):
            z = dot(jnp.concatenate([xt, h], axis=1), wu) + bias
            gi = sigm(z[:, :u])
            gf = sigm(z[:, u:2 * u])
            gg = jnp.tanh(z[:, 2 * u:3 * u])
            go = sigm(z[:, 3 * u:])
            c_new = gf * c + gi * gg
            h_new = go * jnp.tanh(c_new)
            return h_new, c_new

        h = jnp.zeros((block_b, u), jnp.float32)
        c = jnp.zeros((block_b, u), jnp.float32)
        for t in range(t_len):
            h, c = step(x_ref[t], h, c, wf, bf)
            hcat_ref[:, t * 2 * u: t * 2 * u + u] = h

        h = jnp.zeros((block_b, u), jnp.float32)
        c = jnp.zeros((block_b, u), jnp.float32)
        for t in range(t_len - 1, -1, -1):
            h, c = step(x_ref[t], h, c, wb, bb)
            hcat_ref[:, t * 2 * u + u: (t + 1) * 2 * u] = h

        flat = hcat_ref[...]
        d1 = jnp.maximum(dot(flat, w1_ref[...]) + b1_ref[...], 0.0)
        logits = dot(d1, w2_ref[...]) + b2_ref[...]
        m = jnp.max(logits, axis=-1, keepdims=True)
        e = jnp.exp(logits - m)
        out_ref[...] = e / jnp.sum(e, axis=-1, keepdims=True)

    return pl.pallas_call(
        body,
        grid=(n_blocks,),
        in_specs=[
            pl.BlockSpec((t_len, block_b, d), lambda i: (0, i, 0)),
            pl.BlockSpec((d + u, h4), lambda i: (0, 0)),
            pl.BlockSpec((1, h4), lambda i: (0, 0)),
            pl.BlockSpec((d + u, h4), lambda i: (0, 0)),
            pl.BlockSpec((1, h4), lambda i: (0, 0)),
            pl.BlockSpec((t_len * 2 * u, f_dim), lambda i: (0, 0)),
            pl.BlockSpec((1, f_dim), lambda i: (0, 0)),
            pl.BlockSpec((f_dim, c_dim), lambda i: (0, 0)),
            pl.BlockSpec((1, c_dim), lambda i: (0, 0)),
        ],
        out_specs=pl.BlockSpec((block_b, c_dim), lambda i: (i, 0)),
        out_shape=jax.ShapeDtypeStruct((b_full, c_dim), jnp.float32),
        scratch_shapes=[pltpu.VMEM((block_b, t_len * 2 * u), jnp.float32)],
    )(x3, jnp.concatenate([W_f, U_f], axis=0), b_f.reshape(1, -1),
      jnp.concatenate([W_b, U_b], axis=0), b_b.reshape(1, -1),
      W1, b1.reshape(1, -1), W2, b2.reshape(1, -1))


def kernel(inputs, emb_table, W_f, U_f, b_f, W_b, U_b, b_b, W1, b1, W2, b2):
    b, t = inputs.shape
    d = emb_table.shape[1]
    n_chunks = 2
    bc = b // n_chunks
    idx_t = inputs.astype(jnp.int32).T
    outs = []
    for ci in range(n_chunks):
        idx_tm = lax.slice(idx_t, (0, ci * bc), (t, (ci + 1) * bc)).reshape(-1)
        x_flat = _sc_gather_time_major(emb_table, idx_tm)
        x3 = x_flat.reshape(t, bc, d)
        outs.append(_tc_bilstm_classifier(x3, W_f, U_f, b_f, W_b, U_b, b_b,
                                          W1, b1, W2, b2))
    return jnp.concatenate(outs, axis=0)


# speedup vs baseline: 1.0378x; 1.0378x over previous
import functools

import jax
import jax.numpy as jnp
from jax import lax
from jax.experimental import pallas as pl
from jax.experimental.pallas import tpu as pltpu
from jax.experimental.pallas import tpu_sc as plsc


def _sc_gather_time_major(table, idx_tm):
    n_rows, d = idx_tm.shape[0], table.shape[1]
    info = plsc.get_sparse_core_info()
    nc, ns = info.num_cores, info.num_subcores
    nw = nc * ns
    per_w = n_rows // nw
    chunk = 200
    n_iter = per_w // chunk
    mesh = plsc.VectorSubcoreMesh(core_axis_name="c", subcore_axis_name="s")

    @functools.partial(
        pl.kernel,
        mesh=mesh,
        out_type=jax.ShapeDtypeStruct((n_rows, d), jnp.float32),
        scratch_types=[
            pltpu.VMEM((per_w,), jnp.int32),
            pltpu.VMEM((chunk, d), jnp.float32),
            pltpu.VMEM((chunk, d), jnp.float32),
            pltpu.SemaphoreType.DMA,
            pltpu.SemaphoreType.DMA,
            pltpu.SemaphoreType.DMA,
            pltpu.SemaphoreType.DMA,
        ],
    )
    def gather_k(table_hbm, idx_hbm, out_hbm, idx_v, rows0, rows1,
                 g0, g1, w0, w1):
        wid = lax.axis_index("s") * nc + lax.axis_index("c")
        base = wid * per_w
        rows = (rows0, rows1)
        gsem = (g0, g1)
        wsem = (w0, w1)

        pltpu.sync_copy(idx_hbm.at[pl.ds(base, per_w)], idx_v)
        hg = [None, None]
        hw = [None, None]
        hg[0] = pltpu.async_copy(table_hbm.at[idx_v.at[pl.ds(0, chunk)]],
                                 rows[0], gsem[0])
        for i in range(n_iter):
            s = i % 2
            o = 1 - s
            if i + 1 < n_iter:
                if hw[o] is not None:
                    hw[o].wait()
                hg[o] = pltpu.async_copy(
                    table_hbm.at[idx_v.at[pl.ds((i + 1) * chunk, chunk)]],
                    rows[o], gsem[o])
            hg[s].wait()
            hw[s] = pltpu.async_copy(
                rows[s], out_hbm.at[pl.ds(base + i * chunk, chunk)], wsem[s])
        if n_iter > 1:
            hw[(n_iter - 2) % 2].wait()
        hw[(n_iter - 1) % 2].wait()

    return gather_k(table, idx_tm)


def _tc_bilstm_classifier(x3, W_f, U_f, b_f, W_b, U_b, b_b, W1, b1, W2, b2,
                          block_b=512):
    t_len, b_full, d = x3.shape
    u = U_f.shape[0]
    h4 = U_f.shape[1]
    f_dim = W1.shape[1]
    c_dim = W2.shape[1]
    n_blocks = b_full // block_b

    def dot(a, b):
        return jnp.dot(a, b, preferred_element_type=jnp.float32)

    def body(x_ref, wf_ref, bf_ref, wb_ref, bb_ref,
             w1_ref, b1_ref, w2_ref, b2_ref, out_ref, hcat_ref):
        wf = wf_ref[...]
        bf = bf_ref[...]
        wb = wb_ref[...]
        bb = bb_ref[...]

        def sigm(z):
            return 0.5 + 0.5 * jnp.tanh(0.5 * z)

        def step(xt, h, c, wu, bias):
            z = dot(jnp.concatenate([xt, h], axis=1), wu) + bias
            gi = sigm(z[:, :u])
            gf = sigm(z[:, u:2 * u])
            gg = jnp.tanh(z[:, 2 * u:3 * u])
            go = sigm(z[:, 3 * u:])
            c_new = gf * c + gi * gg
            h_new = go * jnp.tanh(c_new)
            return h_new, c_new

        h = jnp.zeros((block_b, u), jnp.float32)
        c = jnp.zeros((block_b, u), jnp.float32)
        for t in range(t_len):
            h, c = step(x_ref[t], h, c, wf, bf)
            hcat_ref[:, t * 2 * u: t * 2 * u + u] = h

        h = jnp.zeros((block_b, u), jnp.float32)
        c = jnp.zeros((block_b, u), jnp.float32)
        for t in range(t_len - 1, -1, -1):
            h, c = step(x_ref[t], h, c, wb, bb)
            hcat_ref[:, t * 2 * u + u: (t + 1) * 2 * u] = h

        flat = hcat_ref[...]
        d1 = jnp.maximum(dot(flat, w1_ref[...]) + b1_ref[...], 0.0)
        logits = dot(d1, w2_ref[...]) + b2_ref[...]
        m = jnp.max(logits, axis=-1, keepdims=True)
        e = jnp.exp(logits - m)
        out_ref[...] = e / jnp.sum(e, axis=-1, keepdims=True)

    return pl.pallas_call(
        body,
        grid=(n_blocks,),
        in_specs=[
            pl.BlockSpec((t_len, block_b, d), lambda i: (0, i, 0)),
            pl.BlockSpec((d + u, h4), lambda i: (0, 0)),
            pl.BlockSpec((1, h4), lambda i: (0, 0)),
            pl.BlockSpec((d + u, h4), lambda i: (0, 0)),
            pl.BlockSpec((1, h4), lambda i: (0, 0)),
            pl.BlockSpec((t_len * 2 * u, f_dim), lambda i: (0, 0)),
            pl.BlockSpec((1, f_dim), lambda i: (0, 0)),
            pl.BlockSpec((f_dim, c_dim), lambda i: (0, 0)),
            pl.BlockSpec((1, c_dim), lambda i: (0, 0)),
        ],
        out_specs=pl.BlockSpec((block_b, c_dim), lambda i: (i, 0)),
        out_shape=jax.ShapeDtypeStruct((b_full, c_dim), jnp.float32),
        scratch_shapes=[pltpu.VMEM((block_b, t_len * 2 * u), jnp.float32)],
    )(x3, jnp.concatenate([W_f, U_f], axis=0), b_f.reshape(1, -1),
      jnp.concatenate([W_b, U_b], axis=0), b_b.reshape(1, -1),
      W1, b1.reshape(1, -1), W2, b2.reshape(1, -1))


def kernel(inputs, emb_table, W_f, U_f, b_f, W_b, U_b, b_b, W1, b1, W2, b2):
    b, t = inputs.shape
    d = emb_table.shape[1]
    splits = (3 * b // 8, 5 * b // 8)
    idx_t = inputs.astype(jnp.int32).T
    outs = []
    start = 0
    for bc in splits:
        idx_tm = lax.slice(idx_t, (0, start), (t, start + bc)).reshape(-1)
        x_flat = _sc_gather_time_major(emb_table, idx_tm)
        x3 = x_flat.reshape(t, bc, d)
        outs.append(_tc_bilstm_classifier(x3, W_f, U_f, b_f, W_b, U_b, b_b,
                                          W1, b1, W2, b2))
        start += bc
    return jnp.concatenate(outs, axis=0)
